# baseline (speedup 1.0000x reference)
import functools

import jax
import jax.numpy as jnp
from jax import lax
from jax.experimental import pallas as pl
from jax.experimental.pallas import tpu as pltpu
from jax.experimental.pallas import tpu_sc as plsc

NU = 50000
NI = 50000
NN = NU + NI
D = 32
E = 1600000
EH = E // 2
CH = 128
NCH = EH // CH
G = 8
NFG = 48
NS = 16
WB = 128
NWB = NU // WB
WB_TAIL = NU - NWB * WB
AW = 16
B = 4096
BR = 256
NBLK = B // BR
INV_T = 5.0
MARGIN = 0.1
L_REG = 1e-4

_mesh = plsc.VectorSubcoreMesh(core_axis_name="c", subcore_axis_name="s")
_sc_params = pltpu.CompilerParams(use_tc_tiling_on_sc=False)


def _zero_wrb(wrb, ncol16):
    zero = jnp.zeros((16,), jnp.float32)

    def _z(i, _):
        wrb[i // ncol16, pl.ds((i % ncol16) * 16, 16)] = zero
        return 0

    lax.fori_loop(0, WB * ncol16, _z, 0)


@functools.partial(
    pl.kernel,
    out_type=jax.ShapeDtypeStruct((NN, AW), jnp.float32),
    mesh=_mesh,
    compiler_params=_sc_params,
    scratch_types=[
        pltpu.VMEM((G, CH), jnp.int32),
        pltpu.VMEM((CH, AW), jnp.float32),
        pltpu.VMEM((WB, AW), jnp.float32),
        pltpu.VMEM_SHARED((NU, AW), jnp.float32),
        pltpu.SemaphoreType.DMA,
    ],
)
def _deg(row2d, out, rowg, ones, wrb, acc, sem):
    cid = lax.axis_index("c")
    sid = lax.axis_index("s")
    _zero_wrb(wrb, AW // 16)

    one = jnp.ones((16,), jnp.float32)

    def _fill(i, _):
        ones[i, pl.ds(0, 16)] = one
        return 0

    lax.fori_loop(0, CH, _fill, 0)

    nwb = 24 + jnp.where(sid < NWB - 24 * NS, 1, 0)

    def _z2(j, _):
        pltpu.sync_copy(wrb, acc.at[pl.ds((sid + j * NS) * WB, WB), :])
        return 0

    lax.fori_loop(0, nwb, _z2, 0)

    @pl.when(sid == NS - 1)
    def _():
        pltpu.sync_copy(wrb.at[pl.ds(0, WB_TAIL), :],
                        acc.at[pl.ds(NWB * WB, WB_TAIL), :])

    plsc.subcore_barrier()

    n = 390 + jnp.where(sid < NCH - 390 * NS, 1, 0)
    base_c = sid * 390 + jnp.minimum(sid, NCH - 390 * NS)
    off = cid * NU

    def _localize(ng):
        def _body(i, _):
            r = i // (CH // 16)
            s = (i % (CH // 16)) * 16
            rowg[r, pl.ds(s, 16)] = rowg[r, pl.ds(s, 16)] - off
            return 0
        lax.fori_loop(0, ng * (CH // 16), _body, 0)

    def _group(gi, _):
        cg = cid * NCH + base_c + gi * G
        pltpu.sync_copy(row2d.at[pl.ds(cg, G), :], rowg)
        _localize(G)
        hs = [pltpu.async_copy(ones, acc.at[rowg.at[j]], sem, add=True)
              for j in range(G)]
        for h in hs:
            h.wait()
        return 0

    lax.fori_loop(0, NFG, _group, 0)

    def _tail(k, _):
        cg = cid * NCH + base_c + NFG * G + k
        pltpu.sync_copy(row2d.at[cg], rowg.at[0])
        _localize(1)
        pltpu.async_copy(ones, acc.at[rowg.at[0]], sem, add=True).wait()
        return 0

    lax.fori_loop(0, n - NFG * G, _tail, 0)
    plsc.subcore_barrier()

    def _wbk(j, _):
        r0 = (sid + j * NS) * WB
        pltpu.sync_copy(acc.at[pl.ds(r0, WB), :], wrb)
        pltpu.sync_copy(wrb, out.at[pl.ds(cid * NU + r0, WB), :])
        return 0

    lax.fori_loop(0, nwb, _wbk, 0)

    @pl.when(sid == NS - 1)
    def _():
        pltpu.sync_copy(acc.at[pl.ds(NWB * WB, WB_TAIL), :],
                        wrb.at[pl.ds(0, WB_TAIL), :])
        pltpu.sync_copy(wrb.at[pl.ds(0, WB_TAIL), :],
                        out.at[pl.ds(cid * NU + NWB * WB, WB_TAIL), :])


def _prescale_body(deg_ref, e0_ref, f0_ref, a_ref, a2_ref):
    d = jnp.maximum(deg_ref[...], 1.0)
    a = lax.rsqrt(d)
    a_ref[...] = a
    a2_ref[...] = a * a
    f0_ref[...] = e0_ref[...] * a[:, :1]


def _prescale(deg16, e0):
    nrows = 1000
    grid = (NN // nrows,)
    return pl.pallas_call(
        _prescale_body,
        grid=grid,
        in_specs=[pl.BlockSpec((nrows, AW), lambda i: (i, 0)),
                  pl.BlockSpec((nrows, D), lambda i: (i, 0))],
        out_specs=[pl.BlockSpec((nrows, D), lambda i: (i, 0)),
                   pl.BlockSpec((nrows, AW), lambda i: (i, 0)),
                   pl.BlockSpec((nrows, AW), lambda i: (i, 0))],
        out_shape=[jax.ShapeDtypeStruct((NN, D), jnp.float32),
                   jax.ShapeDtypeStruct((NN, AW), jnp.float32),
                   jax.ShapeDtypeStruct((NN, AW), jnp.float32)],
    )(deg16, e0)


@functools.partial(
    pl.kernel,
    out_type=jax.ShapeDtypeStruct((NN, D), jnp.float32),
    mesh=_mesh,
    compiler_params=_sc_params,
    scratch_types=[
        pltpu.VMEM((G, CH), jnp.int32),
        pltpu.VMEM((G, CH), jnp.int32),
        pltpu.VMEM((4, CH, D), jnp.float32),
        pltpu.VMEM((WB, D), jnp.float32),
        pltpu.VMEM((WB, AW), jnp.float32),
        pltpu.VMEM_SHARED((NU, D), jnp.float32),
        [pltpu.SemaphoreType.DMA] * 4,
        [pltpu.SemaphoreType.DMA] * 4,
    ],
)
def _prop(row2d, col2d, a216, prev, out, rowg, colg, rows, wrb, a2b, acc,
          gsems, ssems):
    cid = lax.axis_index("c")
    sid = lax.axis_index("s")
    _zero_wrb(wrb, D // 16)

    nwb = 24 + jnp.where(sid < NWB - 24 * NS, 1, 0)

    def _z2(j, _):
        pltpu.sync_copy(wrb, acc.at[pl.ds((sid + j * NS) * WB, WB), :])
        return 0

    lax.fori_loop(0, nwb, _z2, 0)

    @pl.when(sid == NS - 1)
    def _():
        pltpu.sync_copy(wrb.at[pl.ds(0, WB_TAIL), :],
                        acc.at[pl.ds(NWB * WB, WB_TAIL), :])

    plsc.subcore_barrier()

    n = 390 + jnp.where(sid < NCH - 390 * NS, 1, 0)
    base_c = sid * 390 + jnp.minimum(sid, NCH - 390 * NS)
    off = cid * NU

    def _localize(ng):
        def _body(i, _):
            r = i // (CH // 16)
            s = (i % (CH // 16)) * 16
            rowg[r, pl.ds(s, 16)] = rowg[r, pl.ds(s, 16)] - off
            return 0
        lax.fori_loop(0, ng * (CH // 16), _body, 0)

    def _group(gi, _):
        cg = cid * NCH + base_c + gi * G
        pltpu.sync_copy(row2d.at[pl.ds(cg, G), :], rowg)
        pltpu.sync_copy(col2d.at[pl.ds(cg, G), :], colg)
        _localize(G)
        gh = {}
        sh = {}
        gh[0] = pltpu.async_copy(prev.at[colg.at[0]], rows.at[0], gsems[0])
        for j in range(G):
            if j + 1 < G:
                if j + 1 >= 4:
                    sh[j + 1 - 4].wait()
                m = (j + 1) % 4
                gh[j + 1] = pltpu.async_copy(prev.at[colg.at[j + 1]],
                                             rows.at[m], gsems[m])
            gh[j].wait()
            sh[j] = pltpu.async_copy(rows.at[j % 4], acc.at[rowg.at[j]],
                                     ssems[j % 4], add=True)
        for j in range(G - 4, G):
            sh[j].wait()
        return 0

    lax.fori_loop(0, NFG, _group, 0)

    def _tail(k, _):
        cg = cid * NCH + base_c + NFG * G + k
        pltpu.sync_copy(row2d.at[cg], rowg.at[0])
        pltpu.sync_copy(col2d.at[cg], colg.at[0])
        _localize(1)
        pltpu.async_copy(prev.at[colg.at[0]], rows.at[0], gsems[0]).wait()
        pltpu.sync_copy(rows.at[0], acc.at[rowg.at[0]], add=True)
        return 0

    lax.fori_loop(0, n - NFG * G, _tail, 0)
    plsc.subcore_barrier()

    def _scale_wrb(nr):
        def _body(r, _):
            av = a2b[r, pl.ds(0, 16)]
            a2 = av[0]
            wrb[r, pl.ds(0, 16)] = wrb[r, pl.ds(0, 16)] * a2
            wrb[r, pl.ds(16, 16)] = wrb[r, pl.ds(16, 16)] * a2
            return 0
        lax.fori_loop(0, nr, _body, 0)

    def _wbk(j, _):
        r0 = (sid + j * NS) * WB
        g0 = cid * NU + r0
        pltpu.sync_copy(acc.at[pl.ds(r0, WB), :], wrb)
        pltpu.sync_copy(a216.at[pl.ds(g0, WB), :], a2b)
        _scale_wrb(WB)
        pltpu.sync_copy(wrb, out.at[pl.ds(g0, WB), :])
        return 0

    lax.fori_loop(0, nwb, _wbk, 0)

    @pl.when(sid == NS - 1)
    def _():
        r0 = NWB * WB
        g0 = cid * NU + r0
        pltpu.sync_copy(acc.at[pl.ds(r0, WB_TAIL), :],
                        wrb.at[pl.ds(0, WB_TAIL), :])
        pltpu.sync_copy(a216.at[pl.ds(g0, WB_TAIL), :],
                        a2b.at[pl.ds(0, WB_TAIL), :])
        _scale_wrb(WB_TAIL)
        pltpu.sync_copy(wrb.at[pl.ds(0, WB_TAIL), :],
                        out.at[pl.ds(g0, WB_TAIL), :])


_BPT = B // (2 * NS)


@functools.partial(
    pl.kernel,
    out_type=[jax.ShapeDtypeStruct((B, D), jnp.float32) for _ in range(5)],
    mesh=_mesh,
    compiler_params=_sc_params,
    scratch_types=[
        pltpu.VMEM((_BPT,), jnp.int32),
        pltpu.VMEM((_BPT, D), jnp.float32),
        pltpu.VMEM((_BPT, D), jnp.float32),
        pltpu.VMEM((_BPT, D), jnp.float32),
        pltpu.VMEM((_BPT, AW), jnp.float32),
        pltpu.SemaphoreType.DMA,
    ],
)
def _gather_mean(e0, f1, f2, f3, a16, user, positive, negative,
                 user_e, pos_e, ego_u, ego_p, ego_n,
                 idxb, rb, e0b, fsb, ab, sem):
    cid = lax.axis_index("c")
    sid = lax.axis_index("s")
    wid = sid * 2 + cid
    base = wid * _BPT

    def _acc_fs(first):
        def _body(i, _):
            r = i // 2
            s = (i % 2) * 16
            v = rb[r, pl.ds(s, 16)]
            if first:
                fsb[r, pl.ds(s, 16)] = v
            else:
                fsb[r, pl.ds(s, 16)] = fsb[r, pl.ds(s, 16)] + v
            return 0
        lax.fori_loop(0, _BPT * 2, _body, 0)

    def _shift_idx(delta):
        def _body(i, _):
            idxb[pl.ds(i * 16, 16)] = idxb[pl.ds(i * 16, 16)] + delta
            return 0
        lax.fori_loop(0, _BPT // 16, _body, 0)

    def _mean_out(out_ref):
        def _body(r, _):
            av = ab[r, pl.ds(0, 16)]
            a = av[0]
            lo = (e0b[r, pl.ds(0, 16)] + fsb[r, pl.ds(0, 16)] / a) * 0.25
            hi = (e0b[r, pl.ds(16, 16)] + fsb[r, pl.ds(16, 16)] / a) * 0.25
            e0b[r, pl.ds(0, 16)] = lo
            e0b[r, pl.ds(16, 16)] = hi
            return 0
        lax.fori_loop(0, _BPT, _body, 0)
        pltpu.sync_copy(e0b, out_ref.at[pl.ds(base, _BPT), :])

    def _segment(idx_hbm, shift, ego_ref, mean_ref):
        pltpu.sync_copy(idx_hbm.at[pl.ds(base, _BPT)], idxb)
        if shift:
            _shift_idx(NU)
        pltpu.async_copy(e0.at[idxb], e0b, sem).wait()
        pltpu.sync_copy(e0b, ego_ref.at[pl.ds(base, _BPT), :])
        pltpu.async_copy(a16.at[idxb], ab, sem).wait()
        for li, t in enumerate((f1, f2, f3)):
            pltpu.async_copy(t.at[idxb], rb, sem).wait()
            _acc_fs(first=(li == 0))
        _mean_out(mean_ref)

    _segment(user, False, ego_u, user_e)
    _segment(positive, True, ego_p, pos_e)

    pltpu.sync_copy(negative.at[pl.ds(base, _BPT)], idxb)
    _shift_idx(NU)
    pltpu.async_copy(e0.at[idxb], rb, sem).wait()
    pltpu.sync_copy(rb, ego_n.at[pl.ds(base, _BPT), :])


def _loss_body(ue_b, pe_b, ue_f, pe_f, eu, ep, en, reg_ref, na_ref):
    i = pl.program_id(0)

    def _nrm(x):
        n = jnp.maximum(jnp.sqrt(jnp.sum(x * x, axis=1, keepdims=True)), 1e-12)
        return x / n

    e1nb = _nrm(ue_b[...])
    e2nb = _nrm(pe_b[...])
    bfull = _nrm(ue_f[...]) + _nrm(pe_f[...])
    t = lax.dot_general(e1nb, bfull, (((1,), (1,)), ((), ())),
                        preferred_element_type=jnp.float32,
                        precision=lax.Precision.HIGHEST)
    f = jnp.exp(t * INV_T) + jnp.exp(jnp.maximum(t - MARGIN, 0.0) * INV_T)
    tot = jnp.sum(f, axis=1)
    sim = jnp.sum(e1nb * e2nb, axis=1)
    pos = jnp.exp(sim * INV_T) + jnp.exp(jnp.maximum(sim - MARGIN, 0.0) * INV_T)
    part = jnp.sum(-jnp.log(pos / tot + 1e-5))

    @pl.when(i == 0)
    def _():
        na_ref[...] = jnp.zeros((1, 1), jnp.float32)

    na_ref[...] = na_ref[...] + part.reshape(1, 1)

    @pl.when(i == NBLK - 1)
    def _():
        na_ref[...] = na_ref[...] * (1.0 / B)
        reg = (L_REG * 0.5 / B) * (
            jnp.sum(eu[...] ** 2) + jnp.sum(ep[...] ** 2) + jnp.sum(en[...] ** 2))
        reg_ref[...] = reg.reshape(1, 1)


def _loss_tc(ue, pe, eu, ep, en):
    full = pl.BlockSpec((B, D), lambda i: (0, 0))
    blk = pl.BlockSpec((BR, D), lambda i: (i, 0))
    scal = pl.BlockSpec((1, 1), lambda i: (0, 0))
    return pl.pallas_call(
        _loss_body,
        grid=(NBLK,),
        in_specs=[blk, blk, full, full, full, full, full],
        out_specs=[scal, scal],
        out_shape=[jax.ShapeDtypeStruct((1, 1), jnp.float32),
                   jax.ShapeDtypeStruct((1, 1), jnp.float32)],
    )(ue, pe, ue, pe, eu, ep, en)


def kernel(user, positive, negative, edge_index, edge_weight, user_emb_w, item_emb_w):
    e0 = jnp.concatenate([user_emb_w, item_emb_w], axis=0)
    row2d = edge_index[0].reshape(2 * NCH, CH)
    col2d = edge_index[1].reshape(2 * NCH, CH)
    deg16 = _deg(row2d)
    f0, a16, a216 = _prescale(deg16, e0)
    f1 = _prop(row2d, col2d, a216, f0)
    f2 = _prop(row2d, col2d, a216, f1)
    f3 = _prop(row2d, col2d, a216, f2)
    ue, pe, eu, ep, en = _gather_mean(e0, f1, f2, f3, a16,
                                      user, positive, negative)
    reg, na = _loss_tc(ue, pe, eu, ep, en)
    return (reg[0, 0], na[0, 0])


# speedup vs baseline: 1.1386x; 1.1386x over previous
import functools

import jax
import jax.numpy as jnp
from jax import lax
from jax.experimental import pallas as pl
from jax.experimental.pallas import tpu as pltpu
from jax.experimental.pallas import tpu_sc as plsc

NU = 50000
NI = 50000
NN = NU + NI
D = 32
E = 1600000
EH = E // 2
CH = 128
NCH = EH // CH
G = 8
NFG = 48
NS = 16
WB = 128
NWB = NU // WB
WB_TAIL = NU - NWB * WB
B = 4096
BR = 256
NBLK = B // BR
INV_T = 5.0
MARGIN = 0.1
L_REG = 1e-4

_mesh = plsc.VectorSubcoreMesh(core_axis_name="c", subcore_axis_name="s")

_SPMM_SCRATCH = [
    pltpu.VMEM((G, CH), jnp.int32),
    pltpu.VMEM((G, CH), jnp.int32),
    pltpu.VMEM((G, CH), jnp.float32),
    pltpu.VMEM((4, CH, D), jnp.float32),
    pltpu.VMEM((WB, D), jnp.float32),
    pltpu.VMEM_SHARED((NU, D), jnp.float32),
    [pltpu.SemaphoreType.DMA] * 4,
    [pltpu.SemaphoreType.DMA] * 4,
]


def _spmm_core(cid, sid, row2d, col2d, w2d, prev, out, rowg, colg, wg, rows,
               wrb, acc, gsems, ssems):

    zero = jnp.zeros((16,), jnp.float32)

    def _z1(i, _):
        wrb[i // 2, pl.ds((i % 2) * 16, 16)] = zero
        return 0

    lax.fori_loop(0, WB * 2, _z1, 0)

    nwb = 24 + jnp.where(sid < NWB - 24 * NS, 1, 0)

    def _z2(j, _):
        pltpu.sync_copy(wrb, acc.at[pl.ds((sid + j * NS) * WB, WB), :])
        return 0

    lax.fori_loop(0, nwb, _z2, 0)

    @pl.when(sid == NS - 1)
    def _():
        pltpu.sync_copy(wrb.at[pl.ds(0, WB_TAIL), :],
                        acc.at[pl.ds(NWB * WB, WB_TAIL), :])

    plsc.subcore_barrier()

    n = 390 + jnp.where(sid < NCH - 390 * NS, 1, 0)
    base_c = sid * 390 + jnp.minimum(sid, NCH - 390 * NS)
    off = cid * NU

    def _scale(j, p):
        def _body(s, _):
            wv = wg[j, pl.ds(s * 16, 16)]
            for q in range(16):
                wq = wv[q]
                e = s * 16 + q
                rows[p, e, pl.ds(0, 16)] = rows[p, e, pl.ds(0, 16)] * wq
                rows[p, e, pl.ds(16, 16)] = rows[p, e, pl.ds(16, 16)] * wq
            return 0
        lax.fori_loop(0, CH // 16, _body, 0)

    def _localize(ng):
        def _body(i, _):
            r = i // (CH // 16)
            s = (i % (CH // 16)) * 16
            rowg[r, pl.ds(s, 16)] = rowg[r, pl.ds(s, 16)] - off
            return 0
        lax.fori_loop(0, ng * (CH // 16), _body, 0)

    def _group(gi, _):
        cg = cid * NCH + base_c + gi * G
        pltpu.sync_copy(row2d.at[pl.ds(cg, G), :], rowg)
        pltpu.sync_copy(col2d.at[pl.ds(cg, G), :], colg)
        pltpu.sync_copy(w2d.at[pl.ds(cg, G), :], wg)
        _localize(G)
        gh = {}
        sh = {}
        gh[0] = pltpu.async_copy(prev.at[colg.at[0]], rows.at[0], gsems[0])
        for j in range(G):
            if j + 1 < G:
                if j + 1 >= 4:
                    sh[j + 1 - 4].wait()
                m = (j + 1) % 4
                gh[j + 1] = pltpu.async_copy(prev.at[colg.at[j + 1]],
                                             rows.at[m], gsems[m])
            gh[j].wait()
            _scale(j, j % 4)
            sh[j] = pltpu.async_copy(rows.at[j % 4], acc.at[rowg.at[j]],
                                     ssems[j % 4], add=True)
        for j in range(G - 4, G):
            sh[j].wait()
        return 0

    lax.fori_loop(0, NFG, _group, 0)

    def _tail(k, _):
        cg = cid * NCH + base_c + NFG * G + k
        pltpu.sync_copy(row2d.at[cg], rowg.at[0])
        pltpu.sync_copy(col2d.at[cg], colg.at[0])
        pltpu.sync_copy(w2d.at[cg], wg.at[0])
        _localize(1)
        pltpu.async_copy(prev.at[colg.at[0]], rows.at[0], gsems[0]).wait()
        _scale(0, 0)
        pltpu.sync_copy(rows.at[0], acc.at[rowg.at[0]], add=True)
        return 0

    lax.fori_loop(0, n - NFG * G, _tail, 0)
    plsc.subcore_barrier()

    def _wbk(j, _):
        r0 = (sid + j * NS) * WB
        pltpu.sync_copy(acc.at[pl.ds(r0, WB), :], wrb)
        pltpu.sync_copy(wrb, out.at[pl.ds(cid * NU + r0, WB), :])
        return 0

    lax.fori_loop(0, nwb, _wbk, 0)

    @pl.when(sid == NS - 1)
    def _():
        pltpu.sync_copy(acc.at[pl.ds(NWB * WB, WB_TAIL), :],
                        wrb.at[pl.ds(0, WB_TAIL), :])
        pltpu.sync_copy(wrb.at[pl.ds(0, WB_TAIL), :],
                        out.at[pl.ds(cid * NU + NWB * WB, WB_TAIL), :])


@functools.partial(
    pl.kernel,
    out_type=jax.ShapeDtypeStruct((NN, D), jnp.float32),
    mesh=_mesh,
    compiler_params=pltpu.CompilerParams(use_tc_tiling_on_sc=False),
    scratch_types=_SPMM_SCRATCH,
)
def _spmm(row2d, col2d, w2d, prev, out, rowg, colg, wg, rows, wrb, acc,
          gsems, ssems):
    cid = lax.axis_index("c")
    sid = lax.axis_index("s")
    _spmm_core(cid, sid, row2d, col2d, w2d, prev, out, rowg, colg, wg, rows,
               wrb, acc, gsems, ssems)


_BPT = B // NS


@functools.partial(
    pl.kernel,
    out_type=[jax.ShapeDtypeStruct((NN, D), jnp.float32)]
    + [jax.ShapeDtypeStruct((B, D), jnp.float32) for _ in range(5)],
    mesh=_mesh,
    compiler_params=pltpu.CompilerParams(use_tc_tiling_on_sc=False),
    scratch_types=_SPMM_SCRATCH,
)
def _spmm_last(row2d, col2d, w2d, prev, e0, e1, user, positive, negative,
               out, user_e, pos_e, ego_u, ego_p, ego_n,
               rowg, colg, wg, rows, wrb, acc, gsems, ssems):
    cid = lax.axis_index("c")
    sid = lax.axis_index("s")
    _spmm_core(cid, sid, row2d, col2d, w2d, prev, out, rowg, colg, wg, rows,
               wrb, acc, gsems, ssems)
    plsc.subcore_barrier()

    tables = (e0, e1, prev, out)

    def _shift_idx():
        def _body(i, _):
            rowg[0, pl.ds(i * 16, 16)] = rowg[0, pl.ds(i * 16, 16)] + NU
            return 0
        lax.fori_loop(0, CH // 16, _body, 0)

    def _add_rb():
        def _body(i, _):
            r = i // 2
            s = (i % 2) * 16
            rows[0, r, pl.ds(s, 16)] = (rows[0, r, pl.ds(s, 16)]
                                        + rows[1, r, pl.ds(s, 16)])
            return 0
        lax.fori_loop(0, CH * 2, _body, 0)

    def _scale_mean():
        def _body(i, _):
            r = i // 2
            s = (i % 2) * 16
            rows[0, r, pl.ds(s, 16)] = rows[0, r, pl.ds(s, 16)] * 0.25
            return 0
        lax.fori_loop(0, CH * 2, _body, 0)

    def _segment(idx_hbm, shift, ego_ref, mean_ref):
        for hb in range(_BPT // CH):
            base = sid * _BPT + hb * CH
            pltpu.sync_copy(idx_hbm.at[pl.ds(base, CH)], rowg.at[0])
            if shift:
                _shift_idx()
            pltpu.async_copy(tables[0].at[rowg.at[0]], rows.at[0],
                             gsems[0]).wait()
            pltpu.sync_copy(rows.at[0], ego_ref.at[pl.ds(base, CH), :])
            for t in tables[1:]:
                pltpu.async_copy(t.at[rowg.at[0]], rows.at[1],
                                 gsems[1]).wait()
                _add_rb()
            _scale_mean()
            pltpu.sync_copy(rows.at[0], mean_ref.at[pl.ds(base, CH), :])

    @pl.when(cid == 0)
    def _():
        _segment(user, False, ego_u, user_e)

    @pl.when(cid == 1)
    def _():
        _segment(positive, True, ego_p, pos_e)
        for hb in range(_BPT // CH):
            base = sid * _BPT + hb * CH
            pltpu.sync_copy(negative.at[pl.ds(base, CH)], rowg.at[0])
            _shift_idx()
            pltpu.async_copy(e0.at[rowg.at[0]], rows.at[0], gsems[0]).wait()
            pltpu.sync_copy(rows.at[0], ego_n.at[pl.ds(base, CH), :])


def _loss_body(ue_b, pe_b, ue_f, pe_f, eu, ep, en, reg_ref, na_ref):
    i = pl.program_id(0)

    def _nrm(x):
        n = jnp.maximum(jnp.sqrt(jnp.sum(x * x, axis=1, keepdims=True)), 1e-12)
        return x / n

    e1nb = _nrm(ue_b[...])
    e2nb = _nrm(pe_b[...])
    bfull = _nrm(ue_f[...]) + _nrm(pe_f[...])
    t = lax.dot_general(e1nb, bfull, (((1,), (1,)), ((), ())),
                        preferred_element_type=jnp.float32,
                        precision=lax.Precision.HIGHEST)
    f = jnp.exp(t * INV_T) + jnp.exp(jnp.maximum(t - MARGIN, 0.0) * INV_T)
    tot = jnp.sum(f, axis=1)
    sim = jnp.sum(e1nb * e2nb, axis=1)
    pos = jnp.exp(sim * INV_T) + jnp.exp(jnp.maximum(sim - MARGIN, 0.0) * INV_T)
    part = jnp.sum(-jnp.log(pos / tot + 1e-5))

    @pl.when(i == 0)
    def _():
        na_ref[...] = jnp.zeros((1, 1), jnp.float32)

    na_ref[...] = na_ref[...] + part.reshape(1, 1)

    @pl.when(i == NBLK - 1)
    def _():
        na_ref[...] = na_ref[...] * (1.0 / B)
        reg = (L_REG * 0.5 / B) * (
            jnp.sum(eu[...] ** 2) + jnp.sum(ep[...] ** 2) + jnp.sum(en[...] ** 2))
        reg_ref[...] = reg.reshape(1, 1)


def _loss_tc(ue, pe, eu, ep, en):
    full = pl.BlockSpec((B, D), lambda i: (0, 0))
    blk = pl.BlockSpec((BR, D), lambda i: (i, 0))
    scal = pl.BlockSpec((1, 1), lambda i: (0, 0))
    return pl.pallas_call(
        _loss_body,
        grid=(NBLK,),
        in_specs=[blk, blk, full, full, full, full, full],
        out_specs=[scal, scal],
        out_shape=[jax.ShapeDtypeStruct((1, 1), jnp.float32),
                   jax.ShapeDtypeStruct((1, 1), jnp.float32)],
    )(ue, pe, ue, pe, eu, ep, en)


def kernel(user, positive, negative, edge_index, edge_weight, user_emb_w, item_emb_w):
    e0 = jnp.concatenate([user_emb_w, item_emb_w], axis=0)
    row2d = edge_index[0].reshape(2 * NCH, CH)
    col2d = edge_index[1].reshape(2 * NCH, CH)
    w2d = edge_weight.reshape(2 * NCH, CH)
    e1 = _spmm(row2d, col2d, w2d, e0)
    e2 = _spmm(row2d, col2d, w2d, e1)
    _e3, ue, pe, eu, ep, en = _spmm_last(row2d, col2d, w2d, e2, e0, e1,
                                         user, positive, negative)
    reg, na = _loss_tc(ue, pe, eu, ep, en)
    return (reg[0, 0], na[0, 0])


# speedup vs baseline: 1.1452x; 1.0058x over previous
import functools

import jax
import jax.numpy as jnp
from jax import lax
from jax.experimental import pallas as pl
from jax.experimental.pallas import tpu as pltpu
from jax.experimental.pallas import tpu_sc as plsc

NU = 50000
NI = 50000
NN = NU + NI
D = 32
E = 1600000
EH = E // 2
CH = 128
NCH = EH // CH
G = 8
NFG = 48
NS = 16
WB = 128
NWB = NU // WB
WB_TAIL = NU - NWB * WB
B = 4096
BR = 256
NBLK = B // BR
INV_T = 5.0
MARGIN = 0.1
L_REG = 1e-4

_mesh = plsc.VectorSubcoreMesh(core_axis_name="c", subcore_axis_name="s")

_SPMM_SCRATCH = [
    pltpu.VMEM((G, CH), jnp.int32),
    pltpu.VMEM((G, CH), jnp.int32),
    pltpu.VMEM((G, CH), jnp.float32),
    pltpu.VMEM((4, CH, D), jnp.float32),
    pltpu.VMEM((WB, D), jnp.float32),
    pltpu.VMEM_SHARED((NU, D), jnp.float32),
    [pltpu.SemaphoreType.DMA] * 4,
    [pltpu.SemaphoreType.DMA] * 4,
]


def _spmm_core(cid, sid, row2d, col2d, w2d, prev, out, rowg, colg, wg, rows,
               wrb, acc, gsems, ssems):

    zero = jnp.zeros((16,), jnp.float32)

    def _z1(i, _):
        wrb[i // 2, pl.ds((i % 2) * 16, 16)] = zero
        return 0

    lax.fori_loop(0, WB * 2, _z1, 0)

    nwb = 24 + jnp.where(sid < NWB - 24 * NS, 1, 0)

    def _z2(j, _):
        pltpu.sync_copy(wrb, acc.at[pl.ds((sid + j * NS) * WB, WB), :])
        return 0

    lax.fori_loop(0, nwb, _z2, 0)

    @pl.when(sid == NS - 1)
    def _():
        pltpu.sync_copy(wrb.at[pl.ds(0, WB_TAIL), :],
                        acc.at[pl.ds(NWB * WB, WB_TAIL), :])

    plsc.subcore_barrier()

    n = 390 + jnp.where(sid < NCH - 390 * NS, 1, 0)
    base_c = sid * 390 + jnp.minimum(sid, NCH - 390 * NS)
    off = cid * NU

    def _scale(j, p):
        def _body(s, _):
            wv = wg[j, pl.ds(s * 16, 16)]
            for q in range(16):
                wq = wv[q]
                e = s * 16 + q
                rows[p, e, pl.ds(0, 16)] = rows[p, e, pl.ds(0, 16)] * wq
                rows[p, e, pl.ds(16, 16)] = rows[p, e, pl.ds(16, 16)] * wq
            return 0
        lax.fori_loop(0, CH // 16, _body, 0)

    def _localize(ng):
        def _body(i, _):
            r = i // (CH // 16)
            s = (i % (CH // 16)) * 16
            rowg[r, pl.ds(s, 16)] = rowg[r, pl.ds(s, 16)] - off
            return 0
        lax.fori_loop(0, ng * (CH // 16), _body, 0)

    def _group(gi, _):
        cg = cid * NCH + base_c + gi * G
        pltpu.sync_copy(row2d.at[pl.ds(cg, G), :], rowg)
        pltpu.sync_copy(col2d.at[pl.ds(cg, G), :], colg)
        pltpu.sync_copy(w2d.at[pl.ds(cg, G), :], wg)
        _localize(G)
        gh = {}
        sh = {}
        gh[0] = pltpu.async_copy(prev.at[colg.at[0]], rows.at[0], gsems[0])
        for j in range(G):
            if j + 1 < G:
                if j + 1 >= 4:
                    sh[j + 1 - 4].wait()
                m = (j + 1) % 4
                gh[j + 1] = pltpu.async_copy(prev.at[colg.at[j + 1]],
                                             rows.at[m], gsems[m])
            gh[j].wait()
            _scale(j, j % 4)
            sh[j] = pltpu.async_copy(rows.at[j % 4], acc.at[rowg.at[j]],
                                     ssems[j % 4], add=True)
        for j in range(G - 4, G):
            sh[j].wait()
        return 0

    lax.fori_loop(0, NFG, _group, 0)

    def _tail(k, _):
        cg = cid * NCH + base_c + NFG * G + k
        pltpu.sync_copy(row2d.at[cg], rowg.at[0])
        pltpu.sync_copy(col2d.at[cg], colg.at[0])
        pltpu.sync_copy(w2d.at[cg], wg.at[0])
        _localize(1)
        pltpu.async_copy(prev.at[colg.at[0]], rows.at[0], gsems[0]).wait()
        _scale(0, 0)
        pltpu.sync_copy(rows.at[0], acc.at[rowg.at[0]], add=True)
        return 0

    lax.fori_loop(0, n - NFG * G, _tail, 0)
    plsc.subcore_barrier()

    def _wbk(j, _):
        r0 = (sid + j * NS) * WB
        pltpu.sync_copy(acc.at[pl.ds(r0, WB), :], wrb)
        pltpu.sync_copy(wrb, out.at[pl.ds(cid * NU + r0, WB), :])
        return 0

    lax.fori_loop(0, nwb, _wbk, 0)

    @pl.when(sid == NS - 1)
    def _():
        pltpu.sync_copy(acc.at[pl.ds(NWB * WB, WB_TAIL), :],
                        wrb.at[pl.ds(0, WB_TAIL), :])
        pltpu.sync_copy(wrb.at[pl.ds(0, WB_TAIL), :],
                        out.at[pl.ds(cid * NU + NWB * WB, WB_TAIL), :])


@functools.partial(
    pl.kernel,
    out_type=jax.ShapeDtypeStruct((NN, D), jnp.float32),
    mesh=_mesh,
    compiler_params=pltpu.CompilerParams(use_tc_tiling_on_sc=False),
    scratch_types=_SPMM_SCRATCH,
)
def _spmm(row2d, col2d, w2d, prev, out, rowg, colg, wg, rows, wrb, acc,
          gsems, ssems):
    cid = lax.axis_index("c")
    sid = lax.axis_index("s")
    _spmm_core(cid, sid, row2d, col2d, w2d, prev, out, rowg, colg, wg, rows,
               wrb, acc, gsems, ssems)


_BPT = B // NS


@functools.partial(
    pl.kernel,
    out_type=[jax.ShapeDtypeStruct((NN, D), jnp.float32)]
    + [jax.ShapeDtypeStruct((B, D), jnp.float32) for _ in range(5)],
    mesh=_mesh,
    compiler_params=pltpu.CompilerParams(use_tc_tiling_on_sc=False),
    scratch_types=_SPMM_SCRATCH,
)
def _spmm_last(row2d, col2d, w2d, prev, e0, e1, user, positive, negative,
               out, user_e, pos_e, ego_u, ego_p, ego_n,
               rowg, colg, wg, rows, wrb, acc, gsems, ssems):
    cid = lax.axis_index("c")
    sid = lax.axis_index("s")
    _spmm_core(cid, sid, row2d, col2d, w2d, prev, out, rowg, colg, wg, rows,
               wrb, acc, gsems, ssems)
    plsc.subcore_barrier()

    tables = (e0, e1, prev, out)

    def _shift_idx():
        def _body(i, _):
            rowg[0, pl.ds(i * 16, 16)] = rowg[0, pl.ds(i * 16, 16)] + NU
            return 0
        lax.fori_loop(0, CH // 16, _body, 0)

    def _add_rb():
        def _body(i, _):
            r = i // 2
            s = (i % 2) * 16
            rows[0, r, pl.ds(s, 16)] = (rows[0, r, pl.ds(s, 16)]
                                        + rows[1, r, pl.ds(s, 16)])
            return 0
        lax.fori_loop(0, CH * 2, _body, 0)

    def _scale_mean():
        def _body(i, _):
            r = i // 2
            s = (i % 2) * 16
            rows[0, r, pl.ds(s, 16)] = rows[0, r, pl.ds(s, 16)] * 0.25
            return 0
        lax.fori_loop(0, CH * 2, _body, 0)

    def _segment(idx_hbm, shift, ego_ref, mean_ref):
        for hb in range(_BPT // CH):
            base = sid * _BPT + hb * CH
            pltpu.sync_copy(idx_hbm.at[pl.ds(base, CH)], rowg.at[0])
            if shift:
                _shift_idx()
            pltpu.async_copy(tables[0].at[rowg.at[0]], rows.at[0],
                             gsems[0]).wait()
            pltpu.sync_copy(rows.at[0], ego_ref.at[pl.ds(base, CH), :])
            for t in tables[1:]:
                pltpu.async_copy(t.at[rowg.at[0]], rows.at[1],
                                 gsems[1]).wait()
                _add_rb()
            _scale_mean()
            pltpu.sync_copy(rows.at[0], mean_ref.at[pl.ds(base, CH), :])

    @pl.when(cid == 0)
    def _():
        _segment(user, False, ego_u, user_e)

    @pl.when(cid == 1)
    def _():
        _segment(positive, True, ego_p, pos_e)
        for hb in range(_BPT // CH):
            base = sid * _BPT + hb * CH
            pltpu.sync_copy(negative.at[pl.ds(base, CH)], rowg.at[0])
            _shift_idx()
            pltpu.async_copy(e0.at[rowg.at[0]], rows.at[0], gsems[0]).wait()
            pltpu.sync_copy(rows.at[0], ego_n.at[pl.ds(base, CH), :])


def _loss_body(ue_b, pe_b, ue_f, pe_f, eu, ep, en, reg_ref, na_ref):
    i = pl.program_id(0)

    def _nrm(x):
        n = jnp.maximum(jnp.sqrt(jnp.sum(x * x, axis=1, keepdims=True)), 1e-12)
        return x / n

    e1nb = _nrm(ue_b[...])
    e2nb = _nrm(pe_b[...])
    bfull = _nrm(ue_f[...]) + _nrm(pe_f[...])
    t = lax.dot_general(e1nb, bfull, (((1,), (1,)), ((), ())),
                        preferred_element_type=jnp.float32,
                        precision=lax.Precision.HIGHEST)
    cm = jnp.exp(jnp.float32(-MARGIN * INV_T))
    et = jnp.exp(t * INV_T)
    f = et + jnp.maximum(et * cm, 1.0)
    tot = jnp.sum(f, axis=1)
    sim = jnp.sum(e1nb * e2nb, axis=1)
    es = jnp.exp(sim * INV_T)
    pos = es + jnp.maximum(es * cm, 1.0)
    part = jnp.sum(-jnp.log(pos / tot + 1e-5))

    @pl.when(i == 0)
    def _():
        na_ref[...] = jnp.zeros((1, 1), jnp.float32)

    na_ref[...] = na_ref[...] + part.reshape(1, 1)

    @pl.when(i == NBLK - 1)
    def _():
        na_ref[...] = na_ref[...] * (1.0 / B)
        reg = (L_REG * 0.5 / B) * (
            jnp.sum(eu[...] ** 2) + jnp.sum(ep[...] ** 2) + jnp.sum(en[...] ** 2))
        reg_ref[...] = reg.reshape(1, 1)


def _loss_tc(ue, pe, eu, ep, en):
    full = pl.BlockSpec((B, D), lambda i: (0, 0))
    blk = pl.BlockSpec((BR, D), lambda i: (i, 0))
    scal = pl.BlockSpec((1, 1), lambda i: (0, 0))
    return pl.pallas_call(
        _loss_body,
        grid=(NBLK,),
        in_specs=[blk, blk, full, full, full, full, full],
        out_specs=[scal, scal],
        out_shape=[jax.ShapeDtypeStruct((1, 1), jnp.float32),
                   jax.ShapeDtypeStruct((1, 1), jnp.float32)],
    )(ue, pe, ue, pe, eu, ep, en)


def kernel(user, positive, negative, edge_index, edge_weight, user_emb_w, item_emb_w):
    e0 = jnp.concatenate([user_emb_w, item_emb_w], axis=0)
    row2d = edge_index[0].reshape(2 * NCH, CH)
    col2d = edge_index[1].reshape(2 * NCH, CH)
    w2d = edge_weight.reshape(2 * NCH, CH)
    e1 = _spmm(row2d, col2d, w2d, e0)
    e2 = _spmm(row2d, col2d, w2d, e1)
    _e3, ue, pe, eu, ep, en = _spmm_last(row2d, col2d, w2d, e2, e0, e1,
                                         user, positive, negative)
    reg, na = _loss_tc(ue, pe, eu, ep, en)
    return (reg[0, 0], na[0, 0])


# speedup vs baseline: 1.4029x; 1.2250x over previous
import functools

import jax
import jax.numpy as jnp
from jax import lax
from jax.experimental import pallas as pl
from jax.experimental.pallas import tpu as pltpu
from jax.experimental.pallas import tpu_sc as plsc

NU = 50000
NI = 50000
NN = NU + NI
D = 32
E = 1600000
EH = E // 2
CH = 128
NCH = EH // CH
G = 8
NFG = 48
NS = 16
WB = 128
NWB = NU // WB
WB_TAIL = NU - NWB * WB
B = 4096
BR = 256
NBLK = B // BR
INV_T = 5.0
MARGIN = 0.1
L_REG = 1e-4

_mesh = plsc.VectorSubcoreMesh(core_axis_name="c", subcore_axis_name="s")

_SPMM_SCRATCH = [
    pltpu.VMEM((2, G, CH), jnp.int32),
    pltpu.VMEM((2, G, CH), jnp.int32),
    pltpu.VMEM((2, G, CH), jnp.float32),
    pltpu.VMEM((4, CH, D), jnp.float32),
    pltpu.VMEM((WB, D), jnp.float32),
    pltpu.VMEM_SHARED((NU, D), jnp.float32),
    [pltpu.SemaphoreType.DMA] * 4,
    [pltpu.SemaphoreType.DMA] * 4,
    pltpu.SemaphoreType.DMA,
]


def _spmm_core(cid, sid, row2d, col2d, w2d, prev, out, rowg, colg, wg, rows,
               wrb, acc, gsems, ssems, isem):

    zero = jnp.zeros((16,), jnp.float32)

    def _z1(i, _):
        wrb[i // 2, pl.ds((i % 2) * 16, 16)] = zero
        return 0

    lax.fori_loop(0, WB * 2, _z1, 0)

    nwb = 24 + jnp.where(sid < NWB - 24 * NS, 1, 0)

    def _z2(j, _):
        pltpu.sync_copy(wrb, acc.at[pl.ds((sid + j * NS) * WB, WB), :])
        return 0

    lax.fori_loop(0, nwb, _z2, 0)

    @pl.when(sid == NS - 1)
    def _():
        pltpu.sync_copy(wrb.at[pl.ds(0, WB_TAIL), :],
                        acc.at[pl.ds(NWB * WB, WB_TAIL), :])

    plsc.subcore_barrier()

    n = 390 + jnp.where(sid < NCH - 390 * NS, 1, 0)
    base_c = sid * 390 + jnp.minimum(sid, NCH - 390 * NS)
    off = cid * NU

    def _scale(pb, j, p):
        def _body(s, _):
            wv = wg[pb, j, pl.ds(s * 16, 16)]
            for q in range(16):
                wq = wv[q]
                e = s * 16 + q
                rows[p, e, pl.ds(0, 16)] = rows[p, e, pl.ds(0, 16)] * wq
                rows[p, e, pl.ds(16, 16)] = rows[p, e, pl.ds(16, 16)] * wq
            return 0
        lax.fori_loop(0, CH // 16, _body, 0)

    def _localize(pb, ng):
        def _body(i, _):
            r = i // (CH // 16)
            s = (i % (CH // 16)) * 16
            rowg[pb, r, pl.ds(s, 16)] = rowg[pb, r, pl.ds(s, 16)] - off
            return 0
        lax.fori_loop(0, ng * (CH // 16), _body, 0)

    def _issue_idx(gi, pb):
        cg = cid * NCH + base_c + gi * G
        pltpu.async_copy(row2d.at[pl.ds(cg, G), :], rowg.at[pb], isem)
        pltpu.async_copy(col2d.at[pl.ds(cg, G), :], colg.at[pb], isem)
        pltpu.async_copy(w2d.at[pl.ds(cg, G), :], wg.at[pb], isem)

    def _wait_idx(pb):
        pltpu.make_async_copy(row2d.at[pl.ds(0, G), :], rowg.at[pb], isem).wait()
        pltpu.make_async_copy(col2d.at[pl.ds(0, G), :], colg.at[pb], isem).wait()
        pltpu.make_async_copy(w2d.at[pl.ds(0, G), :], wg.at[pb], isem).wait()

    _issue_idx(0, 0)

    def _group(gi, _):
        pb = gi % 2
        _wait_idx(pb)

        @pl.when(gi + 1 < NFG)
        def _():
            _issue_idx(gi + 1, 1 - pb)

        _localize(pb, G)
        gh = {}
        sh = {}
        gh[0] = pltpu.async_copy(prev.at[colg.at[pb, 0]], rows.at[0],
                                 gsems[0])
        for j in range(G):
            if j + 1 < G:
                if j + 1 >= 4:
                    sh[j + 1 - 4].wait()
                m = (j + 1) % 4
                gh[j + 1] = pltpu.async_copy(prev.at[colg.at[pb, j + 1]],
                                             rows.at[m], gsems[m])
            gh[j].wait()
            _scale(pb, j, j % 4)
            sh[j] = pltpu.async_copy(rows.at[j % 4], acc.at[rowg.at[pb, j]],
                                     ssems[j % 4], add=True)
        for j in range(G - 4, G):
            sh[j].wait()
        return 0

    lax.fori_loop(0, NFG, _group, 0)

    def _tail(k, _):
        cg = cid * NCH + base_c + NFG * G + k
        pltpu.sync_copy(row2d.at[cg], rowg.at[0, 0])
        pltpu.sync_copy(col2d.at[cg], colg.at[0, 0])
        pltpu.sync_copy(w2d.at[cg], wg.at[0, 0])
        _localize(0, 1)
        pltpu.async_copy(prev.at[colg.at[0, 0]], rows.at[0], gsems[0]).wait()
        _scale(0, 0, 0)
        pltpu.sync_copy(rows.at[0], acc.at[rowg.at[0, 0]], add=True)
        return 0

    lax.fori_loop(0, n - NFG * G, _tail, 0)
    plsc.subcore_barrier()

    def _wbk(j, _):
        r0 = (sid + j * NS) * WB
        pltpu.sync_copy(acc.at[pl.ds(r0, WB), :], wrb)
        pltpu.sync_copy(wrb, out.at[pl.ds(cid * NU + r0, WB), :])
        return 0

    lax.fori_loop(0, nwb, _wbk, 0)

    @pl.when(sid == NS - 1)
    def _():
        pltpu.sync_copy(acc.at[pl.ds(NWB * WB, WB_TAIL), :],
                        wrb.at[pl.ds(0, WB_TAIL), :])
        pltpu.sync_copy(wrb.at[pl.ds(0, WB_TAIL), :],
                        out.at[pl.ds(cid * NU + NWB * WB, WB_TAIL), :])


@functools.partial(
    pl.kernel,
    out_type=jax.ShapeDtypeStruct((NN, D), jnp.float32),
    mesh=_mesh,
    compiler_params=pltpu.CompilerParams(use_tc_tiling_on_sc=False),
    scratch_types=_SPMM_SCRATCH,
)
def _spmm(row2d, col2d, w2d, prev, out, rowg, colg, wg, rows, wrb, acc,
          gsems, ssems, isem):
    cid = lax.axis_index("c")
    sid = lax.axis_index("s")
    _spmm_core(cid, sid, row2d, col2d, w2d, prev, out, rowg, colg, wg, rows,
               wrb, acc, gsems, ssems, isem)


_BPT = B // NS


@functools.partial(
    pl.kernel,
    out_type=[jax.ShapeDtypeStruct((NN, D), jnp.float32)]
    + [jax.ShapeDtypeStruct((B, D), jnp.float32) for _ in range(5)],
    mesh=_mesh,
    compiler_params=pltpu.CompilerParams(use_tc_tiling_on_sc=False),
    scratch_types=_SPMM_SCRATCH,
)
def _spmm_last(row2d, col2d, w2d, prev, e0, e1, user, positive, negative,
               out, user_e, pos_e, ego_u, ego_p, ego_n,
               rowg, colg, wg, rows, wrb, acc, gsems, ssems, isem):
    cid = lax.axis_index("c")
    sid = lax.axis_index("s")
    _spmm_core(cid, sid, row2d, col2d, w2d, prev, out, rowg, colg, wg, rows,
               wrb, acc, gsems, ssems, isem)
    plsc.subcore_barrier()

    tables = (e0, e1, prev, out)

    def _shift_idx():
        def _body(i, _):
            rowg[0, 0, pl.ds(i * 16, 16)] = rowg[0, 0, pl.ds(i * 16, 16)] + NU
            return 0
        lax.fori_loop(0, CH // 16, _body, 0)

    def _add_rb():
        def _body(i, _):
            r = i // 2
            s = (i % 2) * 16
            rows[0, r, pl.ds(s, 16)] = (rows[0, r, pl.ds(s, 16)]
                                        + rows[1, r, pl.ds(s, 16)])
            return 0
        lax.fori_loop(0, CH * 2, _body, 0)

    def _scale_mean():
        def _body(i, _):
            r = i // 2
            s = (i % 2) * 16
            rows[0, r, pl.ds(s, 16)] = rows[0, r, pl.ds(s, 16)] * 0.25
            return 0
        lax.fori_loop(0, CH * 2, _body, 0)

    def _segment(idx_hbm, shift, ego_ref, mean_ref):
        for hb in range(_BPT // CH):
            base = sid * _BPT + hb * CH
            pltpu.sync_copy(idx_hbm.at[pl.ds(base, CH)], rowg.at[0, 0])
            if shift:
                _shift_idx()
            pltpu.async_copy(tables[0].at[rowg.at[0, 0]], rows.at[0],
                             gsems[0]).wait()
            pltpu.sync_copy(rows.at[0], ego_ref.at[pl.ds(base, CH), :])
            for t in tables[1:]:
                pltpu.async_copy(t.at[rowg.at[0, 0]], rows.at[1],
                                 gsems[1]).wait()
                _add_rb()
            _scale_mean()
            pltpu.sync_copy(rows.at[0], mean_ref.at[pl.ds(base, CH), :])

    @pl.when(cid == 0)
    def _():
        _segment(user, False, ego_u, user_e)

    @pl.when(cid == 1)
    def _():
        _segment(positive, True, ego_p, pos_e)
        for hb in range(_BPT // CH):
            base = sid * _BPT + hb * CH
            pltpu.sync_copy(negative.at[pl.ds(base, CH)], rowg.at[0, 0])
            _shift_idx()
            pltpu.async_copy(e0.at[rowg.at[0, 0]], rows.at[0], gsems[0]).wait()
            pltpu.sync_copy(rows.at[0], ego_n.at[pl.ds(base, CH), :])


def _loss_body(ue_b, pe_b, ue_f, pe_f, eu, ep, en, reg_ref, na_ref):
    i = pl.program_id(0)

    def _nrm(x):
        n = jnp.maximum(jnp.sqrt(jnp.sum(x * x, axis=1, keepdims=True)), 1e-12)
        return x / n

    e1nb = _nrm(ue_b[...])
    e2nb = _nrm(pe_b[...])
    bfull = _nrm(ue_f[...]) + _nrm(pe_f[...])
    t = lax.dot_general(e1nb, bfull, (((1,), (1,)), ((), ())),
                        preferred_element_type=jnp.float32,
                        precision=lax.Precision.HIGHEST)
    cm = jnp.exp(jnp.float32(-MARGIN * INV_T))
    et = jnp.exp(t * INV_T)
    f = et + jnp.maximum(et * cm, 1.0)
    tot = jnp.sum(f, axis=1)
    sim = jnp.sum(e1nb * e2nb, axis=1)
    es = jnp.exp(sim * INV_T)
    pos = es + jnp.maximum(es * cm, 1.0)
    part = jnp.sum(-jnp.log(pos / tot + 1e-5))

    @pl.when(i == 0)
    def _():
        na_ref[...] = jnp.zeros((1, 1), jnp.float32)

    na_ref[...] = na_ref[...] + part.reshape(1, 1)

    @pl.when(i == NBLK - 1)
    def _():
        na_ref[...] = na_ref[...] * (1.0 / B)
        reg = (L_REG * 0.5 / B) * (
            jnp.sum(eu[...] ** 2) + jnp.sum(ep[...] ** 2) + jnp.sum(en[...] ** 2))
        reg_ref[...] = reg.reshape(1, 1)


def _loss_tc(ue, pe, eu, ep, en):
    full = pl.BlockSpec((B, D), lambda i: (0, 0))
    blk = pl.BlockSpec((BR, D), lambda i: (i, 0))
    scal = pl.BlockSpec((1, 1), lambda i: (0, 0))
    return pl.pallas_call(
        _loss_body,
        grid=(NBLK,),
        in_specs=[blk, blk, full, full, full, full, full],
        out_specs=[scal, scal],
        out_shape=[jax.ShapeDtypeStruct((1, 1), jnp.float32),
                   jax.ShapeDtypeStruct((1, 1), jnp.float32)],
    )(ue, pe, ue, pe, eu, ep, en)


def kernel(user, positive, negative, edge_index, edge_weight, user_emb_w, item_emb_w):
    e0 = jnp.concatenate([user_emb_w, item_emb_w], axis=0)
    row2d = edge_index[0].reshape(2 * NCH, CH)
    col2d = edge_index[1].reshape(2 * NCH, CH)
    w2d = edge_weight.reshape(2 * NCH, CH)
    e1 = _spmm(row2d, col2d, w2d, e0)
    e2 = _spmm(row2d, col2d, w2d, e1)
    _e3, ue, pe, eu, ep, en = _spmm_last(row2d, col2d, w2d, e2, e0, e1,
                                         user, positive, negative)
    reg, na = _loss_tc(ue, pe, eu, ep, en)
    return (reg[0, 0], na[0, 0])
